# Initial kernel scaffold; baseline (speedup 1.0000x reference)
import jax
import jax.numpy as jnp
from jax.experimental import pallas as pl


def kernel(user_emb, item_emb, edge_user, edge_item):
    raise NotImplementedError("write your pallas kernel here")


# speedup vs baseline: 30.0242x; 30.0242x over previous
import functools

import jax
import jax.numpy as jnp
from jax import lax
from jax.experimental import pallas as pl
from jax.experimental.pallas import tpu as pltpu
from jax.experimental.pallas import tpu_sc as plsc

NU = 50000
DIM = 32
NC = 2
NS = 16
NP = 50176
ROWS_T = NP // NS
RCH = 64
NCH = ROWS_T // RCH
EB = 64
SBI = 8

_mesh = plsc.VectorSubcoreMesh(
    core_axis_name="c", subcore_axis_name="s", num_cores=NC, num_subcores=NS
)


def _make_stage(n_edge_rows):
    et_rows = n_edge_rows // NS
    nsb = et_rows // SBI

    @functools.partial(
        pl.kernel,
        out_type=jax.ShapeDtypeStruct((NC * NP, DIM), jnp.float32),
        mesh=_mesh,
        compiler_params=pltpu.CompilerParams(
            use_tc_tiling_on_sc=False, needs_layout_passes=False),
        scratch_types=[
            pltpu.VMEM_SHARED((NP, DIM), jnp.float32),
            pltpu.VMEM((SBI, EB), jnp.int32),
            pltpu.VMEM((SBI, EB), jnp.int32),
            pltpu.VMEM((2, EB, DIM), jnp.float32),
            pltpu.VMEM((RCH,), jnp.float32),
            pltpu.SemaphoreType.DMA,
            pltpu.SemaphoreType.DMA,
        ],
    )
    def stage(tab, src, dst, dinv, out, accum, src_buf, dst_buf, rows_buf,
              dinv_buf, sem0, sem1):
        c = lax.axis_index("c")
        s = lax.axis_index("s")
        row0 = s * ROWS_T
        sems = (sem0, sem1)

        def zrow(r, _):
            for w in range(DIM // 16):
                rows_buf[0, r, pl.ds(w * 16, 16)] = jnp.zeros((16,),
                                                              jnp.float32)
            return 0
        lax.fori_loop(0, EB, zrow, 0)
        for k in range(NCH):
            pltpu.sync_copy(rows_buf.at[0],
                            accum.at[pl.ds(row0 + k * RCH, RCH)])
        plsc.subcore_barrier()

        erow0 = c * n_edge_rows + s * et_rows

        def sb_body(sb, _):
            base = erow0 + sb * SBI
            pltpu.sync_copy(src.at[pl.ds(base, SBI)], src_buf)
            pltpu.sync_copy(dst.at[pl.ds(base, SBI)], dst_buf)
            d = pltpu.async_copy(tab.at[src_buf.at[0]], rows_buf.at[0], sem0)
            for j in range(SBI):
                if j + 1 < SBI:
                    d_next = pltpu.async_copy(
                        tab.at[src_buf.at[j + 1]],
                        rows_buf.at[(j + 1) % 2], sems[(j + 1) % 2])
                d.wait()
                pltpu.sync_copy(rows_buf.at[j % 2], accum.at[dst_buf.at[j]],
                                add=True)
                if j + 1 < SBI:
                    d = d_next
            return 0

        lax.fori_loop(0, nsb, sb_body, 0)
        plsc.subcore_barrier()

        for k in range(NCH):
            pltpu.sync_copy(accum.at[pl.ds(row0 + k * RCH, RCH)],
                            rows_buf.at[0])
            pltpu.sync_copy(
                dinv.at[pl.ds(c * NP + row0 + k * RCH, RCH)], dinv_buf)

            def row_body(r, _):
                g = plsc.load_gather(dinv_buf,
                                     [jnp.full((16,), r, jnp.int32)])
                for w in range(DIM // 16):
                    rows_buf[0, r, pl.ds(w * 16, 16)] = (
                        rows_buf[0, r, pl.ds(w * 16, 16)] * g)
                return 0

            lax.fori_loop(0, RCH, row_body, 0)
            pltpu.sync_copy(
                rows_buf.at[0],
                out.at[pl.ds(c * NP + row0 + k * RCH, RCH)])

    return stage


def _make_degree(n_edge_rows):
    et_rows = n_edge_rows // NS
    nsb = et_rows // SBI

    @functools.partial(
        pl.kernel,
        out_type=jax.ShapeDtypeStruct((NC * NP, 16), jnp.float32),
        mesh=_mesh,
        compiler_params=pltpu.CompilerParams(
            use_tc_tiling_on_sc=False, needs_layout_passes=False),
        scratch_types=[
            pltpu.VMEM_SHARED((NP, 16), jnp.float32),
            pltpu.VMEM((SBI, EB), jnp.int32),
            pltpu.VMEM((EB, 16), jnp.float32),
        ],
    )
    def degree(dst, out, dcum, dst_buf, ones_buf):
        c = lax.axis_index("c")
        s = lax.axis_index("s")
        row0 = s * ROWS_T

        def zrow(r, _):
            ones_buf[r, pl.ds(0, 16)] = jnp.zeros((16,), jnp.float32)
            return 0
        lax.fori_loop(0, EB, zrow, 0)
        for k in range(NCH):
            pltpu.sync_copy(ones_buf, dcum.at[pl.ds(row0 + k * RCH, RCH)])
        plsc.subcore_barrier()

        def orow(r, _):
            ones_buf[r, pl.ds(0, 16)] = jnp.ones((16,), jnp.float32)
            return 0
        lax.fori_loop(0, EB, orow, 0)

        erow0 = c * n_edge_rows + s * et_rows

        def sb_body(sb, _):
            base = erow0 + sb * SBI
            pltpu.sync_copy(dst.at[pl.ds(base, SBI)], dst_buf)
            for j in range(SBI):
                pltpu.sync_copy(ones_buf, dcum.at[dst_buf.at[j]], add=True)
            return 0

        lax.fori_loop(0, nsb, sb_body, 0)
        plsc.subcore_barrier()
        pltpu.sync_copy(dcum.at[pl.ds(row0, ROWS_T)],
                        out.at[pl.ds(c * NP + row0, ROWS_T)])

    return degree


def kernel(user_emb, item_emb, edge_user, edge_item):
    E = edge_user.shape[0]
    egran = NS * SBI * EB
    epad = ((E + egran - 1) // egran) * egran
    n_edge_rows = epad // EB

    eu = edge_user.astype(jnp.int32)
    ei = edge_item.astype(jnp.int32)
    pad = jnp.full((epad - E,), NU, jnp.int32)
    eu = jnp.concatenate([eu, pad])
    ei = jnp.concatenate([ei, pad])

    def erows(*parts):
        return jnp.concatenate(parts).reshape(2 * n_edge_rows, EB)

    src_odd = erows(eu, ei + NP)
    dst_odd = erows(ei, eu)
    src_even = erows(ei, eu + NP)
    dst_even = erows(eu, ei)

    degree_k = _make_degree(n_edge_rows)
    stage_k = _make_stage(n_edge_rows)

    deg = degree_k(dst_even)
    deg_u = jnp.where(deg[:NU, 0] == 0, 1.0, deg[:NU, 0])
    deg_i = jnp.where(deg[NP:NP + NU, 0] == 0, 1.0, deg[NP:NP + NU, 0])

    zpadn = jnp.zeros((NP - NU,), jnp.float32)
    dinv_u = jnp.concatenate([1.0 / deg_u, zpadn])
    dinv_i = jnp.concatenate([1.0 / deg_i, zpadn])
    dinv_odd = jnp.concatenate([dinv_i, dinv_u])
    dinv_even = jnp.concatenate([dinv_u, dinv_i])

    isd_u = deg_u ** -0.5
    isd_i = deg_i ** -0.5
    zpadr = jnp.zeros((NP - NU, DIM), jnp.float32)
    x0 = jnp.concatenate([
        user_emb * isd_u[:, None], zpadr,
        item_emb * isd_i[:, None], zpadr,
    ], axis=0)

    y1 = stage_k(x0, src_odd, dst_odd, dinv_odd)
    y2 = stage_k(y1, src_even, dst_even, dinv_even)
    y3 = stage_k(y2, src_odd, dst_odd, dinv_odd)
    y4 = stage_k(y3, src_even, dst_even, dinv_even)

    p1 = y1[:NU]
    q1 = y1[NP:NP + NU]
    p2 = y2[:NU]
    q2 = y2[NP:NP + NU]
    p4 = y4[:NU]
    q4 = y4[NP:NP + NU]

    sd_u = jnp.sqrt(deg_u)[:, None]
    sd_i = jnp.sqrt(deg_i)[:, None]
    final_user = (2.0 * user_emb + sd_u * (q1 + 2.0 * p2 + p4)) / 3.0
    final_item = (2.0 * item_emb + sd_i * (p1 + 2.0 * q2 + q4)) / 3.0
    return (final_user, final_item)


# speedup vs baseline: 35.0078x; 1.1660x over previous
import functools

import jax
import jax.numpy as jnp
from jax import lax
from jax.experimental import pallas as pl
from jax.experimental.pallas import tpu as pltpu
from jax.experimental.pallas import tpu_sc as plsc

NU = 50000
DIM = 32
NC = 2
NS = 16
NP = 50688
ROWS_T = NP // NS
RCH = 96
NCH = ROWS_T // RCH
EB = 96
SBI = 16

_mesh = plsc.VectorSubcoreMesh(
    core_axis_name="c", subcore_axis_name="s", num_cores=NC, num_subcores=NS
)


def _make_stage(n_edge_rows):
    et_rows = n_edge_rows // NS
    nsb = et_rows // SBI

    @functools.partial(
        pl.kernel,
        out_type=jax.ShapeDtypeStruct((NC * NP, DIM), jnp.float32),
        mesh=_mesh,
        compiler_params=pltpu.CompilerParams(
            use_tc_tiling_on_sc=False, needs_layout_passes=False),
        scratch_types=[
            pltpu.VMEM_SHARED((NP, DIM), jnp.float32),
            pltpu.VMEM((SBI, EB), jnp.int32),
            pltpu.VMEM((SBI, EB), jnp.int32),
            pltpu.VMEM((2, EB, DIM), jnp.float32),
            pltpu.VMEM((RCH,), jnp.float32),
            pltpu.SemaphoreType.DMA,
            pltpu.SemaphoreType.DMA,
            pltpu.SemaphoreType.DMA,
            pltpu.SemaphoreType.DMA,
        ],
    )
    def stage(tab, src, dst, dinv, out, accum, src_buf, dst_buf, rows_buf,
              dinv_buf, gsem0, gsem1, ssem0, ssem1):
        c = lax.axis_index("c")
        s = lax.axis_index("s")
        row0 = s * ROWS_T
        gsems = (gsem0, gsem1)
        ssems = (ssem0, ssem1)

        def zrow(r, _):
            for w in range(DIM // 16):
                rows_buf[0, r, pl.ds(w * 16, 16)] = jnp.zeros((16,),
                                                              jnp.float32)
            return 0
        lax.fori_loop(0, EB, zrow, 0)
        for k in range(NCH):
            pltpu.sync_copy(rows_buf.at[0],
                            accum.at[pl.ds(row0 + k * RCH, RCH)])
        plsc.subcore_barrier()

        erow0 = c * n_edge_rows + s * et_rows

        def sb_body(sb, _):
            base = erow0 + sb * SBI
            pltpu.sync_copy(src.at[pl.ds(base, SBI)], src_buf)
            pltpu.sync_copy(dst.at[pl.ds(base, SBI)], dst_buf)
            d = pltpu.async_copy(tab.at[src_buf.at[0]], rows_buf.at[0],
                                 gsem0)
            sdescs = [None] * SBI
            for j in range(SBI):
                slot = j % 2
                if j + 1 < SBI:
                    if j >= 1:
                        sdescs[j - 1].wait()
                    d_next = pltpu.async_copy(
                        tab.at[src_buf.at[j + 1]],
                        rows_buf.at[(j + 1) % 2], gsems[(j + 1) % 2])
                d.wait()
                sd = pltpu.make_async_copy(
                    rows_buf.at[slot], accum.at[dst_buf.at[j]], ssems[slot])
                sd.start(add=True)
                sdescs[j] = sd
                if j + 1 < SBI:
                    d = d_next
            sdescs[SBI - 2].wait()
            sdescs[SBI - 1].wait()
            return 0

        lax.fori_loop(0, nsb, sb_body, 0)
        plsc.subcore_barrier()

        for k in range(NCH):
            pltpu.sync_copy(accum.at[pl.ds(row0 + k * RCH, RCH)],
                            rows_buf.at[0])
            pltpu.sync_copy(
                dinv.at[pl.ds(c * NP + row0 + k * RCH, RCH)], dinv_buf)

            def row_body(r, _):
                g = plsc.load_gather(dinv_buf,
                                     [jnp.full((16,), r, jnp.int32)])
                for w in range(DIM // 16):
                    rows_buf[0, r, pl.ds(w * 16, 16)] = (
                        rows_buf[0, r, pl.ds(w * 16, 16)] * g)
                return 0

            lax.fori_loop(0, RCH, row_body, 0)
            pltpu.sync_copy(
                rows_buf.at[0],
                out.at[pl.ds(c * NP + row0 + k * RCH, RCH)])

    return stage


def _make_degree(n_edge_rows):
    et_rows = n_edge_rows // NS
    nsb = et_rows // SBI

    @functools.partial(
        pl.kernel,
        out_type=jax.ShapeDtypeStruct((NC * NP, 16), jnp.float32),
        mesh=_mesh,
        compiler_params=pltpu.CompilerParams(
            use_tc_tiling_on_sc=False, needs_layout_passes=False),
        scratch_types=[
            pltpu.VMEM_SHARED((NP, 16), jnp.float32),
            pltpu.VMEM((SBI, EB), jnp.int32),
            pltpu.VMEM((EB, 16), jnp.float32),
        ],
    )
    def degree(dst, out, dcum, dst_buf, ones_buf):
        c = lax.axis_index("c")
        s = lax.axis_index("s")
        row0 = s * ROWS_T

        def zrow(r, _):
            ones_buf[r, pl.ds(0, 16)] = jnp.zeros((16,), jnp.float32)
            return 0
        lax.fori_loop(0, EB, zrow, 0)
        for k in range(NCH):
            pltpu.sync_copy(ones_buf, dcum.at[pl.ds(row0 + k * RCH, RCH)])
        plsc.subcore_barrier()

        def orow(r, _):
            ones_buf[r, pl.ds(0, 16)] = jnp.ones((16,), jnp.float32)
            return 0
        lax.fori_loop(0, EB, orow, 0)

        erow0 = c * n_edge_rows + s * et_rows

        def sb_body(sb, _):
            base = erow0 + sb * SBI
            pltpu.sync_copy(dst.at[pl.ds(base, SBI)], dst_buf)
            for j in range(SBI):
                pltpu.sync_copy(ones_buf, dcum.at[dst_buf.at[j]], add=True)
            return 0

        lax.fori_loop(0, nsb, sb_body, 0)
        plsc.subcore_barrier()
        pltpu.sync_copy(dcum.at[pl.ds(row0, ROWS_T)],
                        out.at[pl.ds(c * NP + row0, ROWS_T)])

    return degree


def kernel(user_emb, item_emb, edge_user, edge_item):
    E = edge_user.shape[0]
    egran = NS * SBI * EB
    epad = ((E + egran - 1) // egran) * egran
    n_edge_rows = epad // EB

    eu = edge_user.astype(jnp.int32)
    ei = edge_item.astype(jnp.int32)
    pad = jnp.full((epad - E,), NU, jnp.int32)
    eu = jnp.concatenate([eu, pad])
    ei = jnp.concatenate([ei, pad])

    def erows(*parts):
        return jnp.concatenate(parts).reshape(2 * n_edge_rows, EB)

    src_odd = erows(eu, ei + NP)
    dst_odd = erows(ei, eu)
    src_even = erows(ei, eu + NP)
    dst_even = erows(eu, ei)

    degree_k = _make_degree(n_edge_rows)
    stage_k = _make_stage(n_edge_rows)

    deg = degree_k(dst_even)
    deg_u = jnp.where(deg[:NU, 0] == 0, 1.0, deg[:NU, 0])
    deg_i = jnp.where(deg[NP:NP + NU, 0] == 0, 1.0, deg[NP:NP + NU, 0])

    zpadn = jnp.zeros((NP - NU,), jnp.float32)
    dinv_u = jnp.concatenate([1.0 / deg_u, zpadn])
    dinv_i = jnp.concatenate([1.0 / deg_i, zpadn])
    dinv_odd = jnp.concatenate([dinv_i, dinv_u])
    dinv_even = jnp.concatenate([dinv_u, dinv_i])

    isd_u = deg_u ** -0.5
    isd_i = deg_i ** -0.5
    zpadr = jnp.zeros((NP - NU, DIM), jnp.float32)
    x0 = jnp.concatenate([
        user_emb * isd_u[:, None], zpadr,
        item_emb * isd_i[:, None], zpadr,
    ], axis=0)

    y1 = stage_k(x0, src_odd, dst_odd, dinv_odd)
    y2 = stage_k(y1, src_even, dst_even, dinv_even)
    y3 = stage_k(y2, src_odd, dst_odd, dinv_odd)
    y4 = stage_k(y3, src_even, dst_even, dinv_even)

    p1 = y1[:NU]
    q1 = y1[NP:NP + NU]
    p2 = y2[:NU]
    q2 = y2[NP:NP + NU]
    p4 = y4[:NU]
    q4 = y4[NP:NP + NU]

    sd_u = jnp.sqrt(deg_u)[:, None]
    sd_i = jnp.sqrt(deg_i)[:, None]
    final_user = (2.0 * user_emb + sd_u * (q1 + 2.0 * p2 + p4)) / 3.0
    final_item = (2.0 * item_emb + sd_i * (p1 + 2.0 * q2 + q4)) / 3.0
    return (final_user, final_item)
